# baseline (speedup 1.0000x reference)
import functools
import operator

import jax
import jax.numpy as jnp
from jax import lax
from jax.experimental import pallas as pl
from jax.experimental.pallas import tpu as pltpu
from jax.experimental.pallas import tpu_sc as plsc

N_VERTICES = 1000000
EMBED_DIM = 16
BATCH = 16384
N_NEGS = 20

NC = 2
NS = 16
NW = NC * NS
PER_W = BATCH // NW
CHUNK = 256
GROUPS = CHUNK // 16


def _sc_scores_kernel(tgt_hbm, ctx_hbm, neg_hbm, in_emb, out_emb,
                      pos_out, neg_out,
                      ti, ci, ni, urows, vrows, nrows, possv, negsv, sem):
    wid = lax.axis_index("s") * NC + lax.axis_index("c")
    base = pl.multiple_of(wid * PER_W, CHUNK)

    iota16 = lax.iota(jnp.int32, 16)
    cols = [jnp.full((16,), d, jnp.int32) for d in range(EMBED_DIM)]

    for c in range(PER_W // CHUNK):
        cb = pl.multiple_of(base + c * CHUNK, CHUNK)
        nb = pl.multiple_of(cb * N_NEGS, CHUNK)
        pltpu.sync_copy(tgt_hbm.at[pl.ds(cb, CHUNK)], ti)
        pltpu.sync_copy(ctx_hbm.at[pl.ds(cb, CHUNK)], ci)
        pltpu.sync_copy(neg_hbm.at[pl.ds(nb, CHUNK * N_NEGS)], ni)
        c1 = pltpu.async_copy(in_emb.at[ti], urows, sem)
        c2 = pltpu.async_copy(out_emb.at[ci], vrows, sem)
        c3 = pltpu.async_copy(out_emb.at[ni], nrows, sem)
        c1.wait()
        c2.wait()
        c3.wait()

        def group_body(g, _):
            rows = g * 16 + iota16
            rows20 = rows * N_NEGS
            u_cols = [plsc.load_gather(urows, [rows, cols[d]])
                      for d in range(EMBED_DIM)]
            pos = functools.reduce(
                operator.add,
                [u_cols[d] * plsc.load_gather(vrows, [rows, cols[d]])
                 for d in range(EMBED_DIM)])
            possv[pl.ds(g * 16, 16)] = pos
            for t in range(N_NEGS):
                nr = rows20 + t
                acc = functools.reduce(
                    operator.add,
                    [u_cols[d] * plsc.load_gather(nrows, [nr, cols[d]])
                     for d in range(EMBED_DIM)])
                negsv[pl.ds(t * CHUNK + g * 16, 16)] = acc
            return 0

        lax.fori_loop(0, GROUPS, group_body, 0)

        pltpu.sync_copy(possv, pos_out.at[pl.ds(cb, CHUNK)])
        pltpu.sync_copy(negsv, neg_out.at[pl.ds(nb, CHUNK * N_NEGS)])


def _loss_body(pos_ref, neg_ref, out_ref):
    def logsig(x):
        return jnp.minimum(x, 0.0) - jnp.log1p(jnp.exp(-jnp.abs(x)))

    tot = jnp.sum(logsig(pos_ref[...])) + jnp.sum(logsig(-neg_ref[...]))
    out_ref[0, 0] = -tot / BATCH


@jax.jit
def kernel(target, context, negatives, input_embed, output_embed):
    tgt = target.reshape(-1).astype(jnp.int32)
    ctx = context.reshape(-1).astype(jnp.int32)
    neg = negatives.reshape(-1).astype(jnp.int32)

    mesh = plsc.VectorSubcoreMesh(core_axis_name="c", subcore_axis_name="s",
                                  num_cores=NC, num_subcores=NS)
    sc = pl.kernel(
        _sc_scores_kernel,
        out_type=(jax.ShapeDtypeStruct((BATCH,), jnp.float32),
                  jax.ShapeDtypeStruct((BATCH * N_NEGS,), jnp.float32)),
        mesh=mesh,
        compiler_params=pltpu.CompilerParams(needs_layout_passes=False,
                                             use_tc_tiling_on_sc=False),
        scratch_types=[
            pltpu.VMEM((CHUNK,), jnp.int32),
            pltpu.VMEM((CHUNK,), jnp.int32),
            pltpu.VMEM((CHUNK * N_NEGS,), jnp.int32),
            pltpu.VMEM((CHUNK, EMBED_DIM), jnp.float32),
            pltpu.VMEM((CHUNK, EMBED_DIM), jnp.float32),
            pltpu.VMEM((CHUNK * N_NEGS, EMBED_DIM), jnp.float32),
            pltpu.VMEM((CHUNK,), jnp.float32),
            pltpu.VMEM((CHUNK * N_NEGS,), jnp.float32),
            pltpu.SemaphoreType.DMA,
        ],
    )
    pos_scores, neg_scores = sc(tgt, ctx, neg, input_embed, output_embed)

    loss = pl.pallas_call(
        _loss_body,
        out_shape=jax.ShapeDtypeStruct((1, 1), jnp.float32),
        out_specs=pl.BlockSpec(memory_space=pltpu.SMEM),
    )(pos_scores.reshape(128, 128), neg_scores.reshape(2560, 128))
    return loss[0, 0]
